# baseline (speedup 1.0000x reference)
import functools

import jax
import jax.numpy as jnp
from jax import lax
from jax.experimental import pallas as pl
from jax.experimental.pallas import tpu as pltpu
from jax.experimental.pallas import tpu_sc as plsc

B = 16384
D_NODE = 256
D_MEM = 512
D_EMB = 256

NC = 2
NS = 16
NW = NC * NS
BPW = B // NW
CH = 32
NBUF = 5
NCHUNK = BPW // CH

_mesh = plsc.VectorSubcoreMesh(core_axis_name="c", subcore_axis_name="s")


def _sc_gather_body(nf_hbm, mem_hbm, idx_hbm, gnf_hbm, gmem_hbm,
                    idx_v, nf_buf, mem_buf, *sems):
    wid = lax.axis_index("s") * NC + lax.axis_index("c")
    base = wid * BPW
    pltpu.sync_copy(idx_hbm.at[pl.ds(base, BPW)], idx_v)
    sem_g = sems[:NBUF]
    sem_w = sems[NBUF:]

    def fire_gather(c, p):
        ix = idx_v.at[pl.ds(c * CH, CH)]
        return (pltpu.async_copy(nf_hbm.at[ix], nf_buf.at[p], sem_g[p]),
                pltpu.async_copy(mem_hbm.at[ix], mem_buf.at[p], sem_g[p]))

    def fire_write(c, p):
        o = base + c * CH
        return (pltpu.async_copy(nf_buf.at[p], gnf_hbm.at[pl.ds(o, CH)], sem_w[p]),
                pltpu.async_copy(mem_buf.at[p], gmem_hbm.at[pl.ds(o, CH)], sem_w[p]))

    gather_cps = [None] * NBUF
    write_cps = [None] * NBUF
    for p in range(min(NBUF, NCHUNK)):
        gather_cps[p] = fire_gather(p, p)
    for c in range(NCHUNK):
        p = c % NBUF
        for cp in gather_cps[p]:
            cp.wait()
        write_cps[p] = fire_write(c, p)
        nxt = c + NBUF
        if nxt < NCHUNK:
            for cp in write_cps[p]:
                cp.wait()
            gather_cps[p] = fire_gather(nxt, p)
    for p in range(NBUF):
        if write_cps[p] is not None:
            for cp in write_cps[p]:
                cp.wait()


_sc_gather = functools.partial(
    pl.kernel,
    out_type=(
        jax.ShapeDtypeStruct((B, D_NODE), jnp.float32),
        jax.ShapeDtypeStruct((B, D_MEM), jnp.float32),
    ),
    mesh=_mesh,
    scratch_types=[
        pltpu.VMEM((BPW,), jnp.int32),
        pltpu.VMEM((NBUF, CH, D_NODE), jnp.float32),
        pltpu.VMEM((NBUF, CH, D_MEM), jnp.float32),
    ] + [pltpu.SemaphoreType.DMA] * (2 * NBUF),
)(_sc_gather_body)


TB = 4096


def _mm_body(gnf_ref, gmem_ref, wn_ref, wm_ref, o_ref):
    o_ref[...] = (
        jnp.dot(gnf_ref[...], wn_ref[...], preferred_element_type=jnp.float32)
        + jnp.dot(gmem_ref[...], wm_ref[...], preferred_element_type=jnp.float32)
    )


_mm = pl.pallas_call(
    _mm_body,
    grid=(B // TB,),
    in_specs=[
        pl.BlockSpec((TB, D_NODE), lambda i: (i, 0)),
        pl.BlockSpec((TB, D_MEM), lambda i: (i, 0)),
        pl.BlockSpec((D_NODE, D_EMB), lambda i: (0, 0)),
        pl.BlockSpec((D_MEM, D_EMB), lambda i: (0, 0)),
    ],
    out_specs=pl.BlockSpec((TB, D_EMB), lambda i: (i, 0)),
    out_shape=jax.ShapeDtypeStruct((B, D_EMB), jnp.float32),
)


def kernel(memory, source_nodes, timestamps, node_features,
           W_node, W_mem, W_time, time_w, time_b):
    del timestamps, W_time, time_w, time_b
    gnf, gmem = _sc_gather(node_features, memory, source_nodes)
    return _mm(gnf, gmem, W_node, W_mem)
